# Initial kernel scaffold; baseline (speedup 1.0000x reference)
import functools

import jax
import jax.numpy as jnp
from jax import lax
from jax.experimental import pallas as pl
from jax.experimental.pallas import tpu as pltpu
from jax.experimental.pallas import tpu_sc as plsc

_N = 10000
_E = 320000
_F = 128
_NW = 32
_EPW = _E // _NW
_K = 80
_CH = _EPW // _K
_RPT = _N // 16


def _sc_agg_body(h_hbm, src_hbm, dst_hbm, w_hbm, out0, out1,
                 src_v, dst_v, w_v, rows_v, agg_sh):
    cid = lax.axis_index("c")
    sid = lax.axis_index("s")
    wid = sid * 2 + cid

    zeros = jnp.zeros((16,), jnp.float32)

    def zrow(i, carry):
        for j in range(8):
            rows_v[i, pl.ds(j * 16, 16)] = zeros
        return carry

    lax.fori_loop(0, _K, zrow, 0)
    base = sid * _RPT
    for r in range(_RPT // _K):
        pltpu.sync_copy(rows_v, agg_sh.at[pl.ds(base + r * _K, _K)])
    rem = _RPT % _K
    pltpu.sync_copy(rows_v.at[pl.ds(0, rem)],
                    agg_sh.at[pl.ds(base + (_RPT // _K) * _K, rem)])

    pltpu.sync_copy(src_hbm.at[pl.ds(wid * _CH, _CH)], src_v)
    pltpu.sync_copy(dst_hbm.at[pl.ds(wid * _CH, _CH)], dst_v)
    pltpu.sync_copy(w_hbm.at[pl.ds(wid * _EPW, _EPW)], w_v)
    plsc.subcore_barrier()

    def chunk(c, carry):
        pltpu.sync_copy(h_hbm.at[src_v.at[c]], rows_v)

        def edge(e, c2):
            wsp = plsc.load_gather(
                w_v, [jnp.full((16,), c * _K + e, jnp.int32)])
            for j in range(8):
                sl = pl.ds(j * 16, 16)
                rows_v[e, sl] = rows_v[e, sl] * wsp
            return c2

        lax.fori_loop(0, _K, edge, 0)
        pltpu.sync_copy(rows_v, agg_sh.at[dst_v.at[c]], add=True)
        return carry

    lax.fori_loop(0, _CH, chunk, 0)
    plsc.subcore_barrier()

    @pl.when(cid == 0)
    def _():
        pltpu.sync_copy(agg_sh.at[pl.ds(base, _RPT)],
                        out0.at[pl.ds(base, _RPT)])

    @pl.when(cid == 1)
    def _():
        pltpu.sync_copy(agg_sh.at[pl.ds(base, _RPT)],
                        out1.at[pl.ds(base, _RPT)])


def _sc_agg(h, src2, dst2, w):
    mesh = plsc.VectorSubcoreMesh(core_axis_name="c", subcore_axis_name="s")
    f = pl.kernel(
        _sc_agg_body,
        out_type=[jax.ShapeDtypeStruct((_N, _F), jnp.float32)] * 2,
        mesh=mesh,
        scratch_types=[
            pltpu.VMEM((_CH, _K), jnp.int32),
            pltpu.VMEM((_CH, _K), jnp.int32),
            pltpu.VMEM((_EPW,), jnp.float32),
            pltpu.VMEM((_K, _F), jnp.float32),
            pltpu.VMEM_SHARED((_N, _F), jnp.float32),
        ],
    )
    return f(h, src2, dst2, w)


def _mm_body(a_ref, b_ref, h_ref, wr_ref, wo_ref, bias_ref, o_ref, *, relu):
    agg = a_ref[...] + b_ref[...]
    acc = jnp.dot(agg, wr_ref[...], preferred_element_type=jnp.float32)
    acc = acc + jnp.dot(h_ref[...], wo_ref[...],
                        preferred_element_type=jnp.float32)
    acc = acc + bias_ref[...]
    o_ref[...] = jnp.maximum(acc, 0.0) if relu else acc


def _mm(a, b, h, wr, wo, bias, relu):
    br = 1000
    return pl.pallas_call(
        functools.partial(_mm_body, relu=relu),
        grid=(_N // br,),
        in_specs=[
            pl.BlockSpec((br, _F), lambda i: (i, 0)),
            pl.BlockSpec((br, _F), lambda i: (i, 0)),
            pl.BlockSpec((br, _F), lambda i: (i, 0)),
            pl.BlockSpec((_F, _F), lambda i: (0, 0)),
            pl.BlockSpec((_F, _F), lambda i: (0, 0)),
            pl.BlockSpec((1, _F), lambda i: (0, 0)),
        ],
        out_specs=pl.BlockSpec((br, _F), lambda i: (i, 0)),
        out_shape=jax.ShapeDtypeStruct((_N, _F), jnp.float32),
    )(a, b, h, wr, wo, bias)


def kernel(x, edge_index, edge_weight,
           W1_rel, b1, W1_root, W2_rel, b2, W2_root, W3_rel, b3, W3_root):
    src2 = edge_index[0].reshape(_E // _K, _K)
    dst2 = edge_index[1].reshape(_E // _K, _K)
    h = x
    layers = [(W1_rel, b1, W1_root, True),
              (W2_rel, b2, W2_root, True),
              (W3_rel, b3, W3_root, False)]
    for wr, b, wo, relu in layers:
        p0, p1 = _sc_agg(h, src2, dst2, edge_weight)
        h = _mm(p0, p1, h, wr, wo, b.reshape(1, _F), relu)
    return h


# speedup vs baseline: 6.4345x; 6.4345x over previous
import functools

import jax
import jax.numpy as jnp
from jax import lax
from jax.experimental import pallas as pl
from jax.experimental.pallas import tpu as pltpu
from jax.experimental.pallas import tpu_sc as plsc

_N = 10000
_E = 320000
_F = 128
_NW = 32
_EPW = _E // _NW
_K = 80
_CH = _EPW // _K
_RPT = 624
_REM = _N - 16 * _RPT


def _sc_agg_body(h_hbm, src_hbm, dst_hbm, w_hbm, out0, out1,
                 src_v, dst_v, w_v, rows_v, agg_sh):
    cid = lax.axis_index("c")
    sid = lax.axis_index("s")
    wid = sid * 2 + cid

    zeros = jnp.zeros((16,), jnp.float32)

    def zrow(i, carry):
        for j in range(8):
            rows_v[i, pl.ds(j * 16, 16)] = zeros
        return carry

    lax.fori_loop(0, _K, zrow, 0)
    base = sid * _RPT
    for r in range(_RPT // _K):
        pltpu.sync_copy(rows_v, agg_sh.at[pl.ds(base + r * _K, _K)])
    rem = _RPT % _K
    pltpu.sync_copy(rows_v.at[pl.ds(0, rem)],
                    agg_sh.at[pl.ds(base + (_RPT // _K) * _K, rem)])

    @pl.when(sid == 0)
    def _():
        pltpu.sync_copy(rows_v.at[pl.ds(0, _REM)],
                        agg_sh.at[pl.ds(16 * _RPT, _REM)])

    pltpu.sync_copy(src_hbm.at[pl.ds(wid * _EPW, _EPW)], src_v)
    pltpu.sync_copy(dst_hbm.at[wid], dst_v)
    pltpu.sync_copy(w_hbm.at[pl.ds(wid * _EPW, _EPW)], w_v)
    plsc.subcore_barrier()

    def chunk(c, carry):
        pltpu.sync_copy(h_hbm.at[src_v.at[pl.ds(c * _K, _K)]], rows_v)

        def group(g, c2):
            w16 = w_v[pl.ds(c * _K + g * 16, 16)]
            for l in range(16):
                wsp = jnp.full((16,), w16[l])
                e = g * 16 + l
                for j in range(8):
                    sl = pl.ds(j * 16, 16)
                    rows_v[e, sl] = rows_v[e, sl] * wsp
            return c2

        lax.fori_loop(0, _K // 16, group, 0)
        pltpu.sync_copy(rows_v, agg_sh.at[dst_v.at[c]], add=True)
        return carry

    lax.fori_loop(0, _CH, chunk, 0)
    plsc.subcore_barrier()

    @pl.when(cid == 0)
    def _():
        pltpu.sync_copy(agg_sh.at[pl.ds(base, _RPT)],
                        out0.at[pl.ds(base, _RPT)])

        @pl.when(sid == 0)
        def _():
            pltpu.sync_copy(agg_sh.at[pl.ds(16 * _RPT, _REM)],
                            out0.at[pl.ds(16 * _RPT, _REM)])

    @pl.when(cid == 1)
    def _():
        pltpu.sync_copy(agg_sh.at[pl.ds(base, _RPT)],
                        out1.at[pl.ds(base, _RPT)])

        @pl.when(sid == 0)
        def _():
            pltpu.sync_copy(agg_sh.at[pl.ds(16 * _RPT, _REM)],
                            out1.at[pl.ds(16 * _RPT, _REM)])


def _sc_agg(h, src2, dst2, w):
    mesh = plsc.VectorSubcoreMesh(core_axis_name="c", subcore_axis_name="s")
    f = pl.kernel(
        _sc_agg_body,
        out_type=[jax.ShapeDtypeStruct((_N, _F), jnp.float32)] * 2,
        mesh=mesh,
        scratch_types=[
            pltpu.VMEM((_EPW,), jnp.int32),
            pltpu.VMEM((_CH, _K), jnp.int32),
            pltpu.VMEM((_EPW,), jnp.float32),
            pltpu.VMEM((_K, _F), jnp.float32),
            pltpu.VMEM_SHARED((_N, _F), jnp.float32),
        ],
    )
    return f(h, src2, dst2, w)


def _mm_body(a_ref, b_ref, h_ref, wr_ref, wo_ref, bias_ref, o_ref, *, relu):
    agg = a_ref[...] + b_ref[...]
    acc = jnp.dot(agg, wr_ref[...], preferred_element_type=jnp.float32)
    acc = acc + jnp.dot(h_ref[...], wo_ref[...],
                        preferred_element_type=jnp.float32)
    acc = acc + bias_ref[...]
    o_ref[...] = jnp.maximum(acc, 0.0) if relu else acc


def _mm(a, b, h, wr, wo, bias, relu):
    br = 1000
    return pl.pallas_call(
        functools.partial(_mm_body, relu=relu),
        grid=(_N // br,),
        in_specs=[
            pl.BlockSpec((br, _F), lambda i: (i, 0)),
            pl.BlockSpec((br, _F), lambda i: (i, 0)),
            pl.BlockSpec((br, _F), lambda i: (i, 0)),
            pl.BlockSpec((_F, _F), lambda i: (0, 0)),
            pl.BlockSpec((_F, _F), lambda i: (0, 0)),
            pl.BlockSpec((1, _F), lambda i: (0, 0)),
        ],
        out_specs=pl.BlockSpec((br, _F), lambda i: (i, 0)),
        out_shape=jax.ShapeDtypeStruct((_N, _F), jnp.float32),
    )(a, b, h, wr, wo, bias)


def kernel(x, edge_index, edge_weight,
           W1_rel, b1, W1_root, W2_rel, b2, W2_root, W3_rel, b3, W3_root):
    src2 = edge_index[0]
    dst2 = edge_index[1].reshape(_NW, _CH, _K)
    h = x
    layers = [(W1_rel, b1, W1_root, True),
              (W2_rel, b2, W2_root, True),
              (W3_rel, b3, W3_root, False)]
    for wr, b, wo, relu in layers:
        p0, p1 = _sc_agg(h, src2, dst2, edge_weight)
        h = _mm(p0, p1, h, wr, wo, b.reshape(1, _F), relu)
    return h


# speedup vs baseline: 10.9032x; 1.6945x over previous
import functools

import jax
import jax.numpy as jnp
from jax import lax
from jax.experimental import pallas as pl
from jax.experimental.pallas import tpu as pltpu
from jax.experimental.pallas import tpu_sc as plsc

_N = 10000
_E = 320000
_F = 128
_NW = 32
_EPW = _E // _NW
_K = 80
_CH = _EPW // _K
_RPT = 624
_REM = _N - 16 * _RPT


def _sc_agg_body(h_hbm, src_hbm, dst_hbm, w_hbm, out0, out1,
                 src_v, dst_v, wbuf0, wbuf1, rows_v, rows_v1, agg_sh,
                 gsem0, gsem1, ssem0, ssem1, wsem0, wsem1):
    cid = lax.axis_index("c")
    sid = lax.axis_index("s")
    wid = sid * 2 + cid

    zeros = jnp.zeros((16,), jnp.float32)

    def zrow(i, carry):
        for j in range(8):
            rows_v[i, pl.ds(j * 16, 16)] = zeros
        return carry

    lax.fori_loop(0, _K, zrow, 0)
    base = sid * _RPT
    for r in range(_RPT // _K):
        pltpu.sync_copy(rows_v, agg_sh.at[pl.ds(base + r * _K, _K)])
    rem = _RPT % _K
    pltpu.sync_copy(rows_v.at[pl.ds(0, rem)],
                    agg_sh.at[pl.ds(base + (_RPT // _K) * _K, rem)])

    @pl.when(sid == 0)
    def _():
        pltpu.sync_copy(rows_v.at[pl.ds(0, _REM)],
                        agg_sh.at[pl.ds(16 * _RPT, _REM)])

    pltpu.sync_copy(src_hbm.at[pl.ds(wid * _EPW, _EPW)], src_v)
    pltpu.sync_copy(dst_hbm.at[wid], dst_v)
    plsc.subcore_barrier()

    bufs = (rows_v, rows_v1)
    gsems = (gsem0, gsem1)
    ssems = (ssem0, ssem1)
    wbufs = (wbuf0, wbuf1)
    wsems = (wsem0, wsem1)

    def w_start(c, buf, sem):
        pltpu.async_copy(
            w_hbm.at[pl.ds(wid * _EPW + c * _K, _K)], buf, sem)

    def w_wait(c, buf, sem):
        pltpu.make_async_copy(
            w_hbm.at[pl.ds(wid * _EPW + c * _K, _K)], buf, sem).wait()

    def g_start(c, buf, sem):
        pltpu.async_copy(h_hbm.at[src_v.at[pl.ds(c * _K, _K)]], buf, sem)

    def g_wait(c, buf, sem):
        pltpu.make_async_copy(
            h_hbm.at[src_v.at[pl.ds(c * _K, _K)]], buf, sem).wait()

    def s_start(c, buf, sem):
        pltpu.async_copy(buf, agg_sh.at[dst_v.at[c]], sem, add=True)

    def s_wait(c, buf, sem):
        pltpu.make_async_copy(buf, agg_sh.at[dst_v.at[c]], sem).wait()

    def mult(c, buf, wb):
        def group(g, c2):
            w16 = wb[pl.ds(g * 16, 16)]
            for l in range(16):
                wsp = jnp.full((16,), w16[l])
                e = g * 16 + l
                for j in range(8):
                    sl = pl.ds(j * 16, 16)
                    buf[e, sl] = buf[e, sl] * wsp
            return c2

        lax.fori_loop(0, _K // 16, group, 0)

    g_start(0, bufs[0], gsems[0])
    g_start(1, bufs[1], gsems[1])
    w_start(0, wbufs[0], wsems[0])
    w_start(1, wbufs[1], wsems[1])

    def step2(i, carry):
        c0 = i * 2
        for b in range(2):
            c = c0 + b
            ob = 1 - b

            @pl.when(c > 0)
            def _():
                s_wait(c - 1, bufs[ob], ssems[ob])

                @pl.when(c + 1 < _CH)
                def _():
                    g_start(c + 1, bufs[ob], gsems[ob])

            g_wait(c, bufs[b], gsems[b])
            w_wait(c, wbufs[b], wsems[b])
            mult(c, bufs[b], wbufs[b])
            s_start(c, bufs[b], ssems[b])

            @pl.when(c + 2 < _CH)
            def _():
                w_start(c + 2, wbufs[b], wsems[b])
        return carry

    lax.fori_loop(0, _CH // 2, step2, 0)
    cl = _CH - 1
    s_wait(cl - 1, bufs[1], ssems[1])
    g_wait(cl, bufs[0], gsems[0])
    w_wait(cl, wbufs[0], wsems[0])
    mult(cl, bufs[0], wbufs[0])
    s_start(cl, bufs[0], ssems[0])
    s_wait(cl, bufs[0], ssems[0])
    plsc.subcore_barrier()

    @pl.when(cid == 0)
    def _():
        pltpu.sync_copy(agg_sh.at[pl.ds(base, _RPT)],
                        out0.at[pl.ds(base, _RPT)])

        @pl.when(sid == 0)
        def _():
            pltpu.sync_copy(agg_sh.at[pl.ds(16 * _RPT, _REM)],
                            out0.at[pl.ds(16 * _RPT, _REM)])

    @pl.when(cid == 1)
    def _():
        pltpu.sync_copy(agg_sh.at[pl.ds(base, _RPT)],
                        out1.at[pl.ds(base, _RPT)])

        @pl.when(sid == 0)
        def _():
            pltpu.sync_copy(agg_sh.at[pl.ds(16 * _RPT, _REM)],
                            out1.at[pl.ds(16 * _RPT, _REM)])


def _sc_agg(h, src2, dst2, w):
    mesh = plsc.VectorSubcoreMesh(core_axis_name="c", subcore_axis_name="s")
    f = pl.kernel(
        _sc_agg_body,
        out_type=[jax.ShapeDtypeStruct((_N, _F), jnp.float32)] * 2,
        mesh=mesh,
        scratch_types=[
            pltpu.VMEM((_EPW,), jnp.int32),
            pltpu.VMEM((_CH, _K), jnp.int32),
            pltpu.VMEM((_K,), jnp.float32),
            pltpu.VMEM((_K,), jnp.float32),
            pltpu.VMEM((_K, _F), jnp.float32),
            pltpu.VMEM((_K, _F), jnp.float32),
            pltpu.VMEM_SHARED((_N, _F), jnp.float32),
            pltpu.SemaphoreType.DMA,
            pltpu.SemaphoreType.DMA,
            pltpu.SemaphoreType.DMA,
            pltpu.SemaphoreType.DMA,
            pltpu.SemaphoreType.DMA,
            pltpu.SemaphoreType.DMA,
        ],
    )
    return f(h, src2, dst2, w)


def _mm_body(a_ref, b_ref, h_ref, wr_ref, wo_ref, bias_ref, o_ref, *, relu):
    agg = a_ref[...] + b_ref[...]
    acc = jnp.dot(agg, wr_ref[...], preferred_element_type=jnp.float32)
    acc = acc + jnp.dot(h_ref[...], wo_ref[...],
                        preferred_element_type=jnp.float32)
    acc = acc + bias_ref[...]
    o_ref[...] = jnp.maximum(acc, 0.0) if relu else acc


def _mm(a, b, h, wr, wo, bias, relu):
    br = 1000
    return pl.pallas_call(
        functools.partial(_mm_body, relu=relu),
        grid=(_N // br,),
        in_specs=[
            pl.BlockSpec((br, _F), lambda i: (i, 0)),
            pl.BlockSpec((br, _F), lambda i: (i, 0)),
            pl.BlockSpec((br, _F), lambda i: (i, 0)),
            pl.BlockSpec((_F, _F), lambda i: (0, 0)),
            pl.BlockSpec((_F, _F), lambda i: (0, 0)),
            pl.BlockSpec((1, _F), lambda i: (0, 0)),
        ],
        out_specs=pl.BlockSpec((br, _F), lambda i: (i, 0)),
        out_shape=jax.ShapeDtypeStruct((_N, _F), jnp.float32),
    )(a, b, h, wr, wo, bias)


def kernel(x, edge_index, edge_weight,
           W1_rel, b1, W1_root, W2_rel, b2, W2_root, W3_rel, b3, W3_root):
    src2 = edge_index[0]
    dst2 = edge_index[1].reshape(_NW, _CH, _K)
    h = x
    layers = [(W1_rel, b1, W1_root, True),
              (W2_rel, b2, W2_root, True),
              (W3_rel, b3, W3_root, False)]
    for wr, b, wo, relu in layers:
        p0, p1 = _sc_agg(h, src2, dst2, edge_weight)
        h = _mm(p0, p1, h, wr, wo, b.reshape(1, _F), relu)
    return h
